# baseline (speedup 1.0000x reference)
import functools

import jax
import jax.numpy as jnp
import numpy as np
from jax import lax
from jax.experimental import pallas as pl
from jax.experimental.pallas import tpu as pltpu
from jax.experimental.pallas import tpu_sc as plsc

_ACCEL_FACTOR = 4
_NUM_CENTRAL_LINES = 30
_ZERO_FILL_WIDTH = 368


def _ky_positions():
    center = _ZERO_FILL_WIDTH // 2
    half_width = _NUM_CENTRAL_LINES // 2
    central = np.arange(center - half_width,
                        center + half_width + _NUM_CENTRAL_LINES % 2)
    accel = np.arange(_ZERO_FILL_WIDTH)[::_ACCEL_FACTOR]
    accel = accel[~np.isin(accel, central)]
    return np.sort(np.concatenate([central, accel]))


_KY = _ky_positions()
_NUM_KY = _KY.shape[0]

_SLABS = 32 * 15
_SLAB_ROWS = 320
_NW = 32
_SPW = _SLABS // _NW
_C = 40
_CPS = _SLAB_ROWS // _C
_NCHUNK = _SPW * _CPS
_NBUF = 4

_COL_STARTS = [0, 16, 32, 48, 64, 80, 96, _NUM_KY - 16]


@jax.jit
def _sc_zero_fill(x3d):
    mesh = plsc.VectorSubcoreMesh(core_axis_name="c", subcore_axis_name="s")

    @functools.partial(
        pl.kernel,
        mesh=mesh,
        out_type=jax.ShapeDtypeStruct((_SLABS, _SLAB_ROWS, _ZERO_FILL_WIDTH),
                                      jnp.float32),
        compiler_params=pltpu.CompilerParams(needs_layout_passes=False),
        scratch_types=(
            [pltpu.VMEM((_C, _NUM_KY), jnp.float32)] * _NBUF
            + [pltpu.VMEM((_C, _ZERO_FILL_WIDTH), jnp.float32)] * _NBUF
            + [pltpu.SemaphoreType.DMA] * (2 * _NBUF)
        ),
    )
    def k(x_hbm, out_hbm, *bufs):
        in_v = bufs[:_NBUF]
        out_v = bufs[_NBUF:2 * _NBUF]
        sem_in = bufs[2 * _NBUF:3 * _NBUF]
        sem_out = bufs[3 * _NBUF:4 * _NBUF]
        wid = lax.axis_index("s") * 2 + lax.axis_index("c")
        slab0 = wid * _SPW

        zeros = jnp.zeros((16,), jnp.float32)

        for b in range(_NBUF):
            @plsc.parallel_loop(0, _C, step=1, unroll=2)
            def _(r, b=b):
                for t in range(_ZERO_FILL_WIDTH // 16):
                    out_v[b][r, pl.ds(t * 16, 16)] = zeros

        iota = lax.iota(jnp.int32, 16)

        def ky_of(j):
            return jnp.where(
                j < 43, 4 * j, jnp.where(j < 73, j + 126, 4 * j - 92))

        col_ix = [ky_of(s + iota) for s in _COL_STARTS]

        def in_copy(g, b):
            slab, r0 = slab0 + g // _CPS, (g % _CPS) * _C
            return pltpu.make_async_copy(
                x_hbm.at[slab, pl.ds(r0, _C), :], in_v[b], sem_in[b])

        def out_copy(g, b):
            slab, r0 = slab0 + g // _CPS, (g % _CPS) * _C
            return pltpu.make_async_copy(
                out_v[b], out_hbm.at[slab, pl.ds(r0, _C), :], sem_out[b])

        def scatter_chunk(b):
            @plsc.parallel_loop(0, _C, step=1, unroll=2)
            def _(r):
                rr = jnp.full((16,), r, dtype=jnp.int32)
                for t, s in enumerate(_COL_STARTS):
                    v = in_v[b][r, pl.ds(s, 16)]
                    plsc.store_scatter(out_v[b], [rr, col_ix[t]], v)

        def step(g, b, guard_in=False, guard_out=False):
            gi_ok = (not guard_in) or (g + _NBUF - 1 < _NCHUNK)
            if gi_ok:
                in_copy(g + _NBUF - 1, (b + _NBUF - 1) % _NBUF).start()
            in_copy(g, b).wait()
            if not guard_out or g >= _NBUF:
                out_copy(g - _NBUF, b).wait()
            scatter_chunk(b)
            out_copy(g, b).start()

        for b in range(_NBUF - 1):
            in_copy(b, b).start()
        for g in range(_NBUF):
            step(g, g % _NBUF, guard_in=False, guard_out=True)

        def quad(i, carry):
            for b in range(_NBUF):
                step(_NBUF * i + b, b)
            return carry

        lax.fori_loop(1, _NCHUNK // _NBUF - 1, quad, 0)

        for g in range(_NCHUNK - _NBUF, _NCHUNK):
            step(g, g % _NBUF, guard_in=True, guard_out=False)
        for g in range(_NCHUNK - _NBUF, _NCHUNK):
            out_copy(g, g % _NBUF).wait()

    return k(x3d)


def kernel(undersampled_ksp):
    lead = undersampled_ksp.shape[:-1]
    x3d = undersampled_ksp.reshape(_SLABS, _SLAB_ROWS, _NUM_KY)
    out = _sc_zero_fill(x3d)
    return out.reshape(*lead, _ZERO_FILL_WIDTH)


# speedup vs baseline: 1.0081x; 1.0081x over previous
import functools

import jax
import jax.numpy as jnp
import numpy as np
from jax import lax
from jax.experimental import pallas as pl
from jax.experimental.pallas import tpu as pltpu
from jax.experimental.pallas import tpu_sc as plsc

_ACCEL_FACTOR = 4
_NUM_CENTRAL_LINES = 30
_ZERO_FILL_WIDTH = 368


def _ky_positions():
    center = _ZERO_FILL_WIDTH // 2
    half_width = _NUM_CENTRAL_LINES // 2
    central = np.arange(center - half_width,
                        center + half_width + _NUM_CENTRAL_LINES % 2)
    accel = np.arange(_ZERO_FILL_WIDTH)[::_ACCEL_FACTOR]
    accel = accel[~np.isin(accel, central)]
    return np.sort(np.concatenate([central, accel]))


_KY = _ky_positions()
_NUM_KY = _KY.shape[0]

_SLABS = 32 * 15
_SLAB_ROWS = 320
_NW = 32
_SPW = _SLABS // _NW
_C = 80
_CPS = _SLAB_ROWS // _C
_NCHUNK = _SPW * _CPS
_NBUF = 3

_COL_STARTS = [0, 16, 32, 48, 64, 80, 96, _NUM_KY - 16]


@jax.jit
def _sc_zero_fill(x3d):
    mesh = plsc.VectorSubcoreMesh(core_axis_name="c", subcore_axis_name="s")

    @functools.partial(
        pl.kernel,
        mesh=mesh,
        out_type=jax.ShapeDtypeStruct((_SLABS, _SLAB_ROWS, _ZERO_FILL_WIDTH),
                                      jnp.float32),
        compiler_params=pltpu.CompilerParams(needs_layout_passes=False),
        scratch_types=(
            [pltpu.VMEM((_C, _NUM_KY), jnp.float32)] * _NBUF
            + [pltpu.VMEM((_C, _ZERO_FILL_WIDTH), jnp.float32)] * _NBUF
            + [pltpu.SemaphoreType.DMA] * (2 * _NBUF)
        ),
    )
    def k(x_hbm, out_hbm, *bufs):
        in_v = bufs[:_NBUF]
        out_v = bufs[_NBUF:2 * _NBUF]
        sem_in = bufs[2 * _NBUF:3 * _NBUF]
        sem_out = bufs[3 * _NBUF:4 * _NBUF]
        wid = lax.axis_index("s") * 2 + lax.axis_index("c")
        slab0 = wid * _SPW

        zeros = jnp.zeros((16,), jnp.float32)

        for b in range(_NBUF):
            @plsc.parallel_loop(0, _C, step=1, unroll=2)
            def _(r, b=b):
                for t in range(_ZERO_FILL_WIDTH // 16):
                    out_v[b][r, pl.ds(t * 16, 16)] = zeros

        iota = lax.iota(jnp.int32, 16)

        def ky_of(j):
            return jnp.where(
                j < 43, 4 * j, jnp.where(j < 73, j + 126, 4 * j - 92))

        col_ix = [ky_of(s + iota) for s in _COL_STARTS]

        def in_copy(g, b):
            slab, r0 = slab0 + g // _CPS, (g % _CPS) * _C
            return pltpu.make_async_copy(
                x_hbm.at[slab, pl.ds(r0, _C), :], in_v[b], sem_in[b])

        def out_copy(g, b):
            slab, r0 = slab0 + g // _CPS, (g % _CPS) * _C
            return pltpu.make_async_copy(
                out_v[b], out_hbm.at[slab, pl.ds(r0, _C), :], sem_out[b])

        def scatter_chunk(b):
            @plsc.parallel_loop(0, _C, step=1, unroll=2)
            def _(r):
                rr = jnp.full((16,), r, dtype=jnp.int32)
                for t, s in enumerate(_COL_STARTS):
                    v = in_v[b][r, pl.ds(s, 16)]
                    plsc.store_scatter(out_v[b], [rr, col_ix[t]], v)

        def step(g, b, guard_in=False, guard_out=False):
            gi_ok = (not guard_in) or (g + _NBUF - 1 < _NCHUNK)
            if gi_ok:
                in_copy(g + _NBUF - 1, (b + _NBUF - 1) % _NBUF).start()
            in_copy(g, b).wait()
            if not guard_out or g >= _NBUF:
                out_copy(g - _NBUF, b).wait()
            scatter_chunk(b)
            out_copy(g, b).start()

        for b in range(_NBUF - 1):
            in_copy(b, b).start()
        for g in range(_NBUF):
            step(g, g % _NBUF, guard_in=False, guard_out=True)

        def quad(i, carry):
            for b in range(_NBUF):
                step(_NBUF * i + b, b)
            return carry

        lax.fori_loop(1, _NCHUNK // _NBUF - 1, quad, 0)

        for g in range(_NCHUNK - _NBUF, _NCHUNK):
            step(g, g % _NBUF, guard_in=True, guard_out=False)
        for g in range(_NCHUNK - _NBUF, _NCHUNK):
            out_copy(g, g % _NBUF).wait()

    return k(x3d)


def kernel(undersampled_ksp):
    lead = undersampled_ksp.shape[:-1]
    x3d = undersampled_ksp.reshape(_SLABS, _SLAB_ROWS, _NUM_KY)
    out = _sc_zero_fill(x3d)
    return out.reshape(*lead, _ZERO_FILL_WIDTH)


# speedup vs baseline: 1.0268x; 1.0186x over previous
import functools

import jax
import jax.numpy as jnp
import numpy as np
from jax import lax
from jax.experimental import pallas as pl
from jax.experimental.pallas import tpu as pltpu
from jax.experimental.pallas import tpu_sc as plsc

_ACCEL_FACTOR = 4
_NUM_CENTRAL_LINES = 30
_ZERO_FILL_WIDTH = 368


def _ky_positions():
    center = _ZERO_FILL_WIDTH // 2
    half_width = _NUM_CENTRAL_LINES // 2
    central = np.arange(center - half_width,
                        center + half_width + _NUM_CENTRAL_LINES % 2)
    accel = np.arange(_ZERO_FILL_WIDTH)[::_ACCEL_FACTOR]
    accel = accel[~np.isin(accel, central)]
    return np.sort(np.concatenate([central, accel]))


_KY = _ky_positions()
_NUM_KY = _KY.shape[0]

_SLABS = 32 * 15
_SLAB_ROWS = 320
_NW = 32
_SPW = _SLABS // _NW
_C = 80
_CPS = _SLAB_ROWS // _C
_NCHUNK = _SPW * _CPS
_NBUF = 2

_COL_STARTS = [0, 16, 32, 48, 64, 80, 96, _NUM_KY - 16]


@jax.jit
def _sc_zero_fill(x3d):
    mesh = plsc.VectorSubcoreMesh(core_axis_name="c", subcore_axis_name="s")

    @functools.partial(
        pl.kernel,
        mesh=mesh,
        out_type=jax.ShapeDtypeStruct((_SLABS, _SLAB_ROWS, _ZERO_FILL_WIDTH),
                                      jnp.float32),
        compiler_params=pltpu.CompilerParams(needs_layout_passes=False),
        scratch_types=(
            [pltpu.VMEM((_C, _NUM_KY), jnp.float32)] * _NBUF
            + [pltpu.VMEM((_C, _ZERO_FILL_WIDTH), jnp.float32)] * _NBUF
            + [pltpu.SemaphoreType.DMA] * (2 * _NBUF)
        ),
    )
    def k(x_hbm, out_hbm, *bufs):
        in_v = bufs[:_NBUF]
        out_v = bufs[_NBUF:2 * _NBUF]
        sem_in = bufs[2 * _NBUF:3 * _NBUF]
        sem_out = bufs[3 * _NBUF:4 * _NBUF]
        wid = lax.axis_index("s") * 2 + lax.axis_index("c")
        slab0 = wid * _SPW

        def in_copy(g, b):
            slab, r0 = slab0 + g // _CPS, (g % _CPS) * _C
            return pltpu.make_async_copy(
                x_hbm.at[slab, pl.ds(r0, _C), :], in_v[b], sem_in[b])

        def out_copy(g, b):
            slab, r0 = slab0 + g // _CPS, (g % _CPS) * _C
            return pltpu.make_async_copy(
                out_v[b], out_hbm.at[slab, pl.ds(r0, _C), :], sem_out[b])

        for b in range(_NBUF):
            in_copy(b, b).start()

        zeros = jnp.zeros((16,), jnp.float32)

        for b in range(_NBUF):
            @plsc.parallel_loop(0, _C, step=1, unroll=2)
            def _(r, b=b):
                for t in range(_ZERO_FILL_WIDTH // 16):
                    out_v[b][r, pl.ds(t * 16, 16)] = zeros

        iota = lax.iota(jnp.int32, 16)

        def ky_of(j):
            return jnp.where(
                j < 43, 4 * j, jnp.where(j < 73, j + 126, 4 * j - 92))

        col_ix = [ky_of(s + iota) for s in _COL_STARTS]

        def scatter_chunk(b):
            @plsc.parallel_loop(0, _C, step=1, unroll=2)
            def _(r):
                rr = jnp.full((16,), r, dtype=jnp.int32)
                for t, s in enumerate(_COL_STARTS):
                    v = in_v[b][r, pl.ds(s, 16)]
                    plsc.store_scatter(out_v[b], [rr, col_ix[t]], v)

        def step(g, b, start_in=True, wait_out=True):
            in_copy(g, b).wait()
            if wait_out:
                out_copy(g - _NBUF, b).wait()
            scatter_chunk(b)
            out_copy(g, b).start()
            if start_in:
                in_copy(g + _NBUF, b).start()

        for g in range(_NBUF):
            step(g, g, wait_out=False)

        def ring(i, carry):
            for b in range(_NBUF):
                step(_NBUF * i + b, b)
            return carry

        lax.fori_loop(1, _NCHUNK // _NBUF - 1, ring, 0)

        for g in range(_NCHUNK - _NBUF, _NCHUNK):
            step(g, g % _NBUF, start_in=False)
        for g in range(_NCHUNK - _NBUF, _NCHUNK):
            out_copy(g, g % _NBUF).wait()

    return k(x3d)


def kernel(undersampled_ksp):
    lead = undersampled_ksp.shape[:-1]
    x3d = undersampled_ksp.reshape(_SLABS, _SLAB_ROWS, _NUM_KY)
    out = _sc_zero_fill(x3d)
    return out.reshape(*lead, _ZERO_FILL_WIDTH)


# speedup vs baseline: 1.0303x; 1.0034x over previous
import functools

import jax
import jax.numpy as jnp
import numpy as np
from jax import lax
from jax.experimental import pallas as pl
from jax.experimental.pallas import tpu as pltpu
from jax.experimental.pallas import tpu_sc as plsc

_ACCEL_FACTOR = 4
_NUM_CENTRAL_LINES = 30
_ZERO_FILL_WIDTH = 368


def _ky_positions():
    center = _ZERO_FILL_WIDTH // 2
    half_width = _NUM_CENTRAL_LINES // 2
    central = np.arange(center - half_width,
                        center + half_width + _NUM_CENTRAL_LINES % 2)
    accel = np.arange(_ZERO_FILL_WIDTH)[::_ACCEL_FACTOR]
    accel = accel[~np.isin(accel, central)]
    return np.sort(np.concatenate([central, accel]))


_KY = _ky_positions()
_NUM_KY = _KY.shape[0]

_SLABS = 32 * 15
_SLAB_ROWS = 320
_NW = 32
_SPW = _SLABS // _NW
_C = 80
_CPS = _SLAB_ROWS // _C
_NCHUNK = _SPW * _CPS
_NBUF = 2

_COL_STARTS = [0, 16, 32, 48, 64, 80, 96, _NUM_KY - 16]


@jax.jit
def _sc_zero_fill(x3d):
    mesh = plsc.VectorSubcoreMesh(core_axis_name="c", subcore_axis_name="s")

    @functools.partial(
        pl.kernel,
        mesh=mesh,
        out_type=jax.ShapeDtypeStruct((_SLABS, _SLAB_ROWS, _ZERO_FILL_WIDTH),
                                      jnp.float32),
        compiler_params=pltpu.CompilerParams(
            needs_layout_passes=False,
            disable_bounds_checks=True,
            disable_semaphore_checks=True,
        ),
        scratch_types=(
            [pltpu.VMEM((_C, _NUM_KY), jnp.float32)] * _NBUF
            + [pltpu.VMEM((_C, _ZERO_FILL_WIDTH), jnp.float32)] * _NBUF
            + [pltpu.SemaphoreType.DMA] * (2 * _NBUF)
        ),
    )
    def k(x_hbm, out_hbm, *bufs):
        in_v = bufs[:_NBUF]
        out_v = bufs[_NBUF:2 * _NBUF]
        sem_in = bufs[2 * _NBUF:3 * _NBUF]
        sem_out = bufs[3 * _NBUF:4 * _NBUF]
        wid = lax.axis_index("s") * 2 + lax.axis_index("c")
        slab0 = wid * _SPW

        def in_copy(g, b):
            slab, r0 = slab0 + g // _CPS, (g % _CPS) * _C
            return pltpu.make_async_copy(
                x_hbm.at[slab, pl.ds(r0, _C), :], in_v[b], sem_in[b])

        def out_copy(g, b):
            slab, r0 = slab0 + g // _CPS, (g % _CPS) * _C
            return pltpu.make_async_copy(
                out_v[b], out_hbm.at[slab, pl.ds(r0, _C), :], sem_out[b])

        for b in range(_NBUF):
            in_copy(b, b).start()

        zeros = jnp.zeros((16,), jnp.float32)

        for b in range(_NBUF):
            @plsc.parallel_loop(0, _C, step=1, unroll=2)
            def _(r, b=b):
                for t in range(_ZERO_FILL_WIDTH // 16):
                    out_v[b][r, pl.ds(t * 16, 16)] = zeros

        iota = lax.iota(jnp.int32, 16)

        def ky_of(j):
            return jnp.where(
                j < 43, 4 * j, jnp.where(j < 73, j + 126, 4 * j - 92))

        col_ix = [ky_of(s + iota) for s in _COL_STARTS]

        def scatter_chunk(b):
            @plsc.parallel_loop(0, _C, step=1, unroll=2)
            def _(r):
                rr = jnp.full((16,), r, dtype=jnp.int32)
                for t, s in enumerate(_COL_STARTS):
                    v = in_v[b][r, pl.ds(s, 16)]
                    plsc.store_scatter(out_v[b], [rr, col_ix[t]], v)

        def step(g, b, start_in=True, wait_out=True):
            in_copy(g, b).wait()
            if wait_out:
                out_copy(g - _NBUF, b).wait()
            scatter_chunk(b)
            out_copy(g, b).start()
            if start_in:
                in_copy(g + _NBUF, b).start()

        for g in range(_NBUF):
            step(g, g, wait_out=False)

        def ring(i, carry):
            for b in range(_NBUF):
                step(_NBUF * i + b, b)
            return carry

        lax.fori_loop(1, _NCHUNK // _NBUF - 1, ring, 0)

        for g in range(_NCHUNK - _NBUF, _NCHUNK):
            step(g, g % _NBUF, start_in=False)
        for g in range(_NCHUNK - _NBUF, _NCHUNK):
            out_copy(g, g % _NBUF).wait()

    return k(x3d)


def kernel(undersampled_ksp):
    lead = undersampled_ksp.shape[:-1]
    x3d = undersampled_ksp.reshape(_SLABS, _SLAB_ROWS, _NUM_KY)
    out = _sc_zero_fill(x3d)
    return out.reshape(*lead, _ZERO_FILL_WIDTH)
